# Initial kernel scaffold; baseline (speedup 1.0000x reference)
import jax
import jax.numpy as jnp
from jax.experimental import pallas as pl


def kernel(z_real, z_imag, W):
    raise NotImplementedError("write your pallas kernel here")


# speedup vs baseline: 1.1703x; 1.1703x over previous
import functools

import jax
import jax.numpy as jnp
from jax import lax
from jax.experimental import pallas as pl
from jax.experimental.pallas import tpu as pltpu
from jax.experimental.pallas import tpu_sc as plsc

N, DIM, K = 4096, 32, 8192
D2 = 2 * DIM
BN = 512
BK = 2048

NC, NS, L = 2, 16, 16
NW = NC * NS
CHUNK = N // NW


def _argmin_body(z_ref, wt_ref, x2_ref, w2_ref, idx_ref):
    z = z_ref[...]
    x2 = x2_ref[...]
    best_d = jnp.full((BN, 1), jnp.inf, jnp.float32)
    best_i = jnp.zeros((BN, 1), jnp.int32)
    for j in range(K // BK):
        wt = wt_ref[:, j * BK:(j + 1) * BK]
        w2 = w2_ref[:, j * BK:(j + 1) * BK]
        s = jnp.dot(z, wt, preferred_element_type=jnp.float32)
        d2 = x2 + w2 - 2.0 * s
        d = jnp.sqrt(jnp.maximum(d2, 0.0))
        m = jnp.min(d, axis=1, keepdims=True)
        ii = jnp.where(d == m,
                       lax.broadcasted_iota(jnp.int32, (BN, BK), 1) + j * BK,
                       K)
        mi = jnp.min(ii, axis=1, keepdims=True)
        upd = m < best_d
        best_i = jnp.where(upd, mi, best_i)
        best_d = jnp.where(upd, m, best_d)
    idx_ref[...] = best_i


def _tc_argmin(z_flat, wt, x2, w2):
    return pl.pallas_call(
        _argmin_body,
        grid=(N // BN,),
        in_specs=[
            pl.BlockSpec((BN, D2), lambda i: (i, 0)),
            pl.BlockSpec((D2, K), lambda i: (0, 0)),
            pl.BlockSpec((BN, 1), lambda i: (i, 0)),
            pl.BlockSpec((1, K), lambda i: (0, 0)),
        ],
        out_specs=pl.BlockSpec((BN, 1), lambda i: (i, 0)),
        out_shape=jax.ShapeDtypeStruct((N, 1), jnp.int32),
    )(z_flat, wt, x2, w2)


def _sc_body(idx_hbm, w_hbm, zq_hbm, counts_hbm,
             idx_v, rows_v, ones_v, zseg_v, counts_sp, sem):
    c = lax.axis_index("c")
    s = lax.axis_index("s")
    wid = s * NC + c
    base = wid * CHUNK

    pltpu.sync_copy(idx_hbm.at[pl.ds(base, CHUNK)], idx_v)
    gather = pltpu.async_copy(w_hbm.at[idx_v], rows_v, sem)

    seg = K // NS
    for i in range(seg // L):
        zseg_v[pl.ds(i * L, L)] = jnp.zeros((L,), jnp.float32)
    pltpu.sync_copy(zseg_v, counts_sp.at[pl.ds(s * seg, seg)])
    for i in range(CHUNK // L):
        ones_v[pl.ds(i * L, L)] = jnp.full((L,), 1.0, jnp.float32)
    plsc.subcore_barrier()

    pltpu.sync_copy(ones_v, counts_sp.at[idx_v], add=True)
    plsc.subcore_barrier()

    @pl.when(s == 0)
    def _():
        pltpu.sync_copy(counts_sp, counts_hbm.at[c])

    gather.wait()
    pltpu.sync_copy(rows_v, zq_hbm.at[pl.ds(base, CHUNK)])


def _sc_gather_hist(indices, w_pad):
    kern = functools.partial(
        pl.kernel,
        mesh=plsc.VectorSubcoreMesh(core_axis_name="c", subcore_axis_name="s"),
        out_type=(
            jax.ShapeDtypeStruct((N, 128), jnp.float32),
            jax.ShapeDtypeStruct((NC, K), jnp.float32),
        ),
        scratch_types=[
            pltpu.VMEM((CHUNK,), jnp.int32),
            pltpu.VMEM((CHUNK, 128), jnp.float32),
            pltpu.VMEM((CHUNK,), jnp.float32),
            pltpu.VMEM((K // NS,), jnp.float32),
            pltpu.VMEM_SHARED((K,), jnp.float32),
            pltpu.SemaphoreType.DMA,
        ],
    )(_sc_body)
    return kern(indices, w_pad)


def _loss_body(zq_ref, z_ref, counts_ref, zst_ref, loss_ref, perp_ref):
    zq = zq_ref[:, :D2]
    z = z_ref[...]
    diff = zq - z
    m = jnp.sum(diff * diff, keepdims=True) * (1.0 / (N * D2))
    loss_ref[...] = m + 0.25 * m
    zst_ref[...] = z + (zq - z)
    p = jnp.sum(counts_ref[...], axis=0, keepdims=True) * (1.0 / N)
    plogp = p * jnp.log(p + 1e-10)
    perp_ref[...] = jnp.exp(-jnp.sum(plogp, axis=1, keepdims=True))


def _tc_epilogue(zq_pad, z_flat, counts):
    return pl.pallas_call(
        _loss_body,
        out_shape=(
            jax.ShapeDtypeStruct((N, D2), jnp.float32),
            jax.ShapeDtypeStruct((1, 1), jnp.float32),
            jax.ShapeDtypeStruct((1, 1), jnp.float32),
        ),
    )(zq_pad, z_flat, counts)


def kernel(z_real, z_imag, W):
    z_flat = jnp.concatenate([z_real, z_imag], axis=-1)
    x2 = jnp.sum(z_flat * z_flat, axis=-1, keepdims=True)
    w2 = jnp.sum(W * W, axis=-1)[None, :]
    idx2d = _tc_argmin(z_flat, W.T, x2, w2)
    indices = jnp.reshape(idx2d, (N,))
    w_pad = jnp.concatenate([W, jnp.zeros((K, 128 - D2), jnp.float32)], axis=1)
    zq_pad, counts = _sc_gather_hist(indices, w_pad)
    zst, loss, perp = _tc_epilogue(zq_pad, z_flat, counts)
    z_q_c = lax.complex(zst[:, :DIM], zst[:, DIM:])
    return z_q_c, jnp.reshape(loss, ()), jnp.reshape(perp, ())


# speedup vs baseline: 1.7600x; 1.5039x over previous
import functools

import jax
import jax.numpy as jnp
from jax import lax
from jax.experimental import pallas as pl
from jax.experimental.pallas import tpu as pltpu
from jax.experimental.pallas import tpu_sc as plsc

N, DIM, K = 4096, 32, 8192
D2 = 2 * DIM
BN = 1024
BK = 2048

NC, NS, L = 2, 16, 16
NW = NC * NS
CHUNK = N // NW


def _argmin_body(z_ref, w_ref, x2_ref, w2_ref, idx_ref, wpad_ref):
    z = z_ref[...]
    zm2 = z * (-2.0)
    x2 = x2_ref[...]
    iota_f = lax.broadcasted_iota(jnp.int32, (BN, BK), 1).astype(jnp.float32)
    best_d = jnp.full((BN, 1), jnp.inf, jnp.float32)
    best_i = jnp.zeros((BN, 1), jnp.float32)
    for j in range(K // BK):
        w = w_ref[j * BK:(j + 1) * BK, :]
        w2 = w2_ref[:, j * BK:(j + 1) * BK]
        s = lax.dot_general(zm2, w, (((1,), (1,)), ((), ())),
                            preferred_element_type=jnp.float32)
        d2 = (x2 + w2) + s
        d = d2 * lax.rsqrt(d2)
        m = jnp.min(d, axis=1, keepdims=True)
        ii = jnp.where(d == m, iota_f, 65536.0)
        mi = jnp.min(ii, axis=1, keepdims=True)
        upd = m < best_d
        best_i = jnp.where(upd, mi + jnp.float32(j * BK), best_i)
        best_d = jnp.where(upd, m, best_d)
    idx_ref[...] = best_i.astype(jnp.int32)

    KS = K // (N // BN)
    wpad_ref[:, :D2] = w_ref[pl.ds(pl.program_id(0) * KS, KS), :]


def _tc_argmin(z_flat, W, x2, w2):
    return pl.pallas_call(
        _argmin_body,
        grid=(N // BN,),
        in_specs=[
            pl.BlockSpec((BN, D2), lambda i: (i, 0)),
            pl.BlockSpec((K, D2), lambda i: (0, 0)),
            pl.BlockSpec((BN, 1), lambda i: (i, 0)),
            pl.BlockSpec((1, K), lambda i: (0, 0)),
        ],
        out_specs=[
            pl.BlockSpec((BN, 1), lambda i: (i, 0)),
            pl.BlockSpec((K // (N // BN), 128), lambda i: (i, 0)),
        ],
        out_shape=[
            jax.ShapeDtypeStruct((N, 1), jnp.int32),
            jax.ShapeDtypeStruct((K, 128), jnp.float32),
        ],
    )(z_flat, W, x2, w2)


def _sc_body(idx_hbm, w_hbm, zq_hbm, counts_hbm,
             idx_v, rows_v, ones_v, zseg_v, counts_sp, sem):
    c = lax.axis_index("c")
    s = lax.axis_index("s")
    wid = s * NC + c
    base = wid * CHUNK

    pltpu.sync_copy(idx_hbm.at[pl.ds(base, CHUNK)], idx_v)
    gather = pltpu.async_copy(w_hbm.at[idx_v], rows_v, sem)

    seg = K // NS
    for i in range(seg // L):
        zseg_v[pl.ds(i * L, L)] = jnp.zeros((L,), jnp.float32)
    pltpu.sync_copy(zseg_v, counts_sp.at[pl.ds(s * seg, seg)])
    for i in range(CHUNK // L):
        ones_v[pl.ds(i * L, L)] = jnp.full((L,), 1.0, jnp.float32)
    plsc.subcore_barrier()

    pltpu.sync_copy(ones_v, counts_sp.at[idx_v], add=True)
    plsc.subcore_barrier()

    @pl.when(s == 0)
    def _():
        pltpu.sync_copy(counts_sp, counts_hbm.at[c])

    gather.wait()
    pltpu.sync_copy(rows_v, zq_hbm.at[pl.ds(base, CHUNK)])


def _sc_gather_hist(indices, w_pad):
    kern = functools.partial(
        pl.kernel,
        mesh=plsc.VectorSubcoreMesh(core_axis_name="c", subcore_axis_name="s"),
        out_type=(
            jax.ShapeDtypeStruct((N, 128), jnp.float32),
            jax.ShapeDtypeStruct((NC, K), jnp.float32),
        ),
        scratch_types=[
            pltpu.VMEM((CHUNK,), jnp.int32),
            pltpu.VMEM((CHUNK, 128), jnp.float32),
            pltpu.VMEM((CHUNK,), jnp.float32),
            pltpu.VMEM((K // NS,), jnp.float32),
            pltpu.VMEM_SHARED((K,), jnp.float32),
            pltpu.SemaphoreType.DMA,
        ],
    )(_sc_body)
    return kern(indices, w_pad)


def _loss_body(zq_ref, z_ref, counts_ref, re_ref, im_ref, loss_ref, perp_ref):
    zq = zq_ref[:, :D2]
    z = z_ref[...]
    diff = zq - z
    m = jnp.sum(diff * diff, keepdims=True) * (1.0 / (N * D2))
    loss_ref[...] = m + 0.25 * m
    zst = z + (zq - z)
    re_ref[...] = zst[:, :DIM]
    im_ref[...] = zst[:, DIM:]
    p = jnp.sum(counts_ref[...], axis=0, keepdims=True) * (1.0 / N)
    plogp = p * jnp.log(p + 1e-10)
    perp_ref[...] = jnp.exp(-jnp.sum(plogp, axis=1, keepdims=True))


def _tc_epilogue(zq_pad, z_flat, counts):
    return pl.pallas_call(
        _loss_body,
        out_shape=(
            jax.ShapeDtypeStruct((N, DIM), jnp.float32),
            jax.ShapeDtypeStruct((N, DIM), jnp.float32),
            jax.ShapeDtypeStruct((1, 1), jnp.float32),
            jax.ShapeDtypeStruct((1, 1), jnp.float32),
        ),
    )(zq_pad, z_flat, counts)


def kernel(z_real, z_imag, W):
    z_flat = jnp.concatenate([z_real, z_imag], axis=-1)
    x2 = jnp.sum(z_flat * z_flat, axis=-1, keepdims=True)
    w2 = jnp.sum(W * W, axis=-1)[None, :]
    idx2d, w_pad = _tc_argmin(z_flat, W, x2, w2)
    indices = jnp.reshape(idx2d, (N,))
    zq_pad, counts = _sc_gather_hist(indices, w_pad)
    re, im, loss, perp = _tc_epilogue(zq_pad, z_flat, counts)
    z_q_c = lax.complex(re, im)
    return z_q_c, jnp.reshape(loss, ()), jnp.reshape(perp, ())


# speedup vs baseline: 1.7729x; 1.0073x over previous
import functools

import jax
import jax.numpy as jnp
from jax import lax
from jax.experimental import pallas as pl
from jax.experimental.pallas import tpu as pltpu
from jax.experimental.pallas import tpu_sc as plsc

N, DIM, K = 4096, 32, 8192
D2 = 2 * DIM
BN = 2048
BK = 2048

NC, NS, L = 2, 16, 16
NW = NC * NS
CHUNK = N // NW


def _argmin_body(z_ref, w_ref, x2_ref, w2_ref, idx_ref, wpad_ref):
    z = z_ref[...]
    zm2 = z * (-2.0)
    x2 = x2_ref[...]
    iota_f = lax.broadcasted_iota(jnp.int32, (BN, BK), 1).astype(jnp.float32)
    best_d = jnp.full((BN, 1), jnp.inf, jnp.float32)
    best_i = jnp.zeros((BN, 1), jnp.float32)
    for j in range(K // BK):
        w = w_ref[j * BK:(j + 1) * BK, :]
        w2 = w2_ref[:, j * BK:(j + 1) * BK]
        s = lax.dot_general(zm2, w, (((1,), (1,)), ((), ())),
                            preferred_element_type=jnp.float32)
        d2 = (x2 + w2) + s
        d = d2 * lax.rsqrt(d2)
        m = jnp.min(d, axis=1, keepdims=True)
        ii = jnp.where(d == m, iota_f, 65536.0)
        mi = jnp.min(ii, axis=1, keepdims=True)
        upd = m < best_d
        best_i = jnp.where(upd, mi + jnp.float32(j * BK), best_i)
        best_d = jnp.where(upd, m, best_d)
    idx_ref[...] = best_i.astype(jnp.int32)

    KS = K // (N // BN)
    wpad_ref[:, :D2] = w_ref[pl.ds(pl.program_id(0) * KS, KS), :]


def _tc_argmin(z_flat, W, x2, w2):
    return pl.pallas_call(
        _argmin_body,
        grid=(N // BN,),
        in_specs=[
            pl.BlockSpec((BN, D2), lambda i: (i, 0)),
            pl.BlockSpec((K, D2), lambda i: (0, 0)),
            pl.BlockSpec((BN, 1), lambda i: (i, 0)),
            pl.BlockSpec((1, K), lambda i: (0, 0)),
        ],
        out_specs=[
            pl.BlockSpec((BN, 1), lambda i: (i, 0)),
            pl.BlockSpec((K // (N // BN), 128), lambda i: (i, 0)),
        ],
        out_shape=[
            jax.ShapeDtypeStruct((N, 1), jnp.int32),
            jax.ShapeDtypeStruct((K, 128), jnp.float32),
        ],
    )(z_flat, W, x2, w2)


def _sc_body(idx_hbm, w_hbm, zq_hbm, counts_hbm,
             idx_v, rows_v, ones_v, zseg_v, counts_sp, sem):
    c = lax.axis_index("c")
    s = lax.axis_index("s")
    wid = s * NC + c
    base = wid * CHUNK

    pltpu.sync_copy(idx_hbm.at[pl.ds(base, CHUNK)], idx_v)
    gather = pltpu.async_copy(w_hbm.at[idx_v], rows_v, sem)

    seg = K // NS
    for i in range(seg // L):
        zseg_v[pl.ds(i * L, L)] = jnp.zeros((L,), jnp.float32)
    pltpu.sync_copy(zseg_v, counts_sp.at[pl.ds(s * seg, seg)])
    for i in range(CHUNK // L):
        ones_v[pl.ds(i * L, L)] = jnp.full((L,), 1.0, jnp.float32)
    plsc.subcore_barrier()

    pltpu.sync_copy(ones_v, counts_sp.at[idx_v], add=True)
    plsc.subcore_barrier()

    @pl.when(s == 0)
    def _():
        pltpu.sync_copy(counts_sp, counts_hbm.at[c])

    gather.wait()
    pltpu.sync_copy(rows_v, zq_hbm.at[pl.ds(base, CHUNK)])


def _sc_gather_hist(indices, w_pad):
    kern = functools.partial(
        pl.kernel,
        mesh=plsc.VectorSubcoreMesh(core_axis_name="c", subcore_axis_name="s"),
        out_type=(
            jax.ShapeDtypeStruct((N, 128), jnp.float32),
            jax.ShapeDtypeStruct((NC, K), jnp.float32),
        ),
        scratch_types=[
            pltpu.VMEM((CHUNK,), jnp.int32),
            pltpu.VMEM((CHUNK, 128), jnp.float32),
            pltpu.VMEM((CHUNK,), jnp.float32),
            pltpu.VMEM((K // NS,), jnp.float32),
            pltpu.VMEM_SHARED((K,), jnp.float32),
            pltpu.SemaphoreType.DMA,
        ],
    )(_sc_body)
    return kern(indices, w_pad)


def _loss_body(zq_ref, z_ref, counts_ref, re_ref, im_ref, loss_ref, perp_ref):
    zq = zq_ref[:, :D2]
    z = z_ref[...]
    diff = zq - z
    m = jnp.sum(diff * diff, keepdims=True) * (1.0 / (N * D2))
    loss_ref[...] = m + 0.25 * m
    zst = z + (zq - z)
    re_ref[...] = zst[:, :DIM]
    im_ref[...] = zst[:, DIM:]
    p = jnp.sum(counts_ref[...], axis=0, keepdims=True) * (1.0 / N)
    plogp = p * jnp.log(p + 1e-10)
    perp_ref[...] = jnp.exp(-jnp.sum(plogp, axis=1, keepdims=True))


def _tc_epilogue(zq_pad, z_flat, counts):
    return pl.pallas_call(
        _loss_body,
        out_shape=(
            jax.ShapeDtypeStruct((N, DIM), jnp.float32),
            jax.ShapeDtypeStruct((N, DIM), jnp.float32),
            jax.ShapeDtypeStruct((1, 1), jnp.float32),
            jax.ShapeDtypeStruct((1, 1), jnp.float32),
        ),
    )(zq_pad, z_flat, counts)


def kernel(z_real, z_imag, W):
    z_flat = jnp.concatenate([z_real, z_imag], axis=-1)
    x2 = jnp.sum(z_flat * z_flat, axis=-1, keepdims=True)
    w2 = jnp.sum(W * W, axis=-1)[None, :]
    idx2d, w_pad = _tc_argmin(z_flat, W, x2, w2)
    indices = jnp.reshape(idx2d, (N,))
    zq_pad, counts = _sc_gather_hist(indices, w_pad)
    re, im, loss, perp = _tc_epilogue(zq_pad, z_flat, counts)
    z_q_c = lax.complex(re, im)
    return z_q_c, jnp.reshape(loss, ()), jnp.reshape(perp, ())
